# baseline (speedup 1.0000x reference)
import functools

import jax
import jax.numpy as jnp
from jax import lax
from jax.experimental import pallas as pl
from jax.experimental.pallas import tpu as pltpu
from jax.experimental.pallas import tpu_sc as plsc

N = 10000
E = 320000
D = 128

NC = 2
NS = 16
NW = NC * NS

NP = 10240
ROWS_W = NP // NW
EPW = E // NW
K = 80
NCHUNK = EPW // K
NPA = 10240
NROWS_T = NPA // NS

_mesh = plsc.VectorSubcoreMesh(core_axis_name="c", subcore_axis_name="s",
                               num_cores=NC, num_subcores=NS)
_cp = pltpu.CompilerParams(needs_layout_passes=False)


def _wid():
    return lax.axis_index("s") * NC + lax.axis_index("c")


@functools.partial(
    pl.kernel,
    out_type=[jax.ShapeDtypeStruct((NP, D), jnp.float32),
              jax.ShapeDtypeStruct((NW, NPA), jnp.float32)],
    mesh=_mesh,
    compiler_params=_cp,
    scratch_types=[
        pltpu.VMEM((K,), jnp.int32),
        pltpu.VMEM((K, D), jnp.float32),
        pltpu.VMEM((NCHUNK, K), jnp.int32),
        pltpu.VMEM((NPA,), jnp.float32),
        pltpu.SemaphoreType.DMA,
    ],
)
def _emb_gather(xp_hbm, emb_hbm, dst3_hbm, zn_hbm, h_out, c_out,
                idx_v, rows_v, didx_all, cnt_v, sem):
    w = _wid()
    base = w * ROWS_W

    def body(i, _):
        off = base + i * K
        pltpu.sync_copy(xp_hbm.at[pl.ds(off, K)], idx_v)
        pltpu.async_copy(emb_hbm.at[idx_v], rows_v, sem).wait()
        pltpu.sync_copy(rows_v, h_out.at[pl.ds(off, K)])
        return 0

    lax.fori_loop(0, ROWS_W // K, body, 0)

    pltpu.sync_copy(zn_hbm, cnt_v)
    pltpu.sync_copy(dst3_hbm.at[w], didx_all)
    ones = jnp.full((16,), 1.0, jnp.float32)

    def cbody(i, _):
        for j in range(K // 16):
            plsc.addupdate_scatter(cnt_v, [didx_all[i, pl.ds(j * 16, 16)]],
                                   ones)
        return 0

    lax.fori_loop(0, NCHUNK, cbody, 0)
    pltpu.sync_copy(cnt_v, c_out.at[w])


@functools.partial(
    pl.kernel,
    out_type=jax.ShapeDtypeStruct((NC, NPA, D), jnp.float32),
    mesh=_mesh,
    compiler_params=_cp,
    scratch_types=(
        [pltpu.VMEM((K,), jnp.int32) for _ in range(3)]
        + [pltpu.VMEM((NCHUNK, K), jnp.int32)]
        + [pltpu.VMEM((K, D), jnp.float32) for _ in range(3)]
        + [pltpu.VMEM_SHARED((NPA, D), jnp.float32)]
        + [pltpu.SemaphoreType.DMA for _ in range(9)]
    ),
)
def _segsum(h_hbm, src_hbm, dst3_hbm, z128_hbm, s_out, *rest):
    core = lax.axis_index("c")
    sub = lax.axis_index("s")
    rbase = sub * NROWS_T
    sidx = rest[0:3]
    didx_all = rest[3]
    rows = rest[4:7]
    acc_sh = rest[7]
    gsem = rest[8:11]
    ssem = rest[11:14]
    isem = rest[14:17]
    pltpu.sync_copy(z128_hbm.at[pl.ds(rbase, NROWS_T)],
                    acc_sh.at[pl.ds(rbase, NROWS_T)])
    w = _wid()
    pltpu.sync_copy(dst3_hbm.at[w], didx_all)
    plsc.subcore_barrier()

    ebase = w * EPW

    def i_start(ci, b):
        pltpu.async_copy(src_hbm.at[pl.ds(ebase + ci * K, K)], sidx[b],
                         isem[b])

    def i_wait(ci, b):
        pltpu.make_async_copy(src_hbm.at[pl.ds(ebase + ci * K, K)], sidx[b],
                              isem[b]).wait()

    def g_start(ci, b):
        pltpu.async_copy(h_hbm.at[sidx[b]], rows[b], gsem[b])

    def g_wait(ci, b):
        pltpu.make_async_copy(h_hbm.at[sidx[b]], rows[b], gsem[b]).wait()

    def s_start(ci, b):
        pltpu.async_copy(rows[b], acc_sh.at[didx_all.at[ci]], ssem[b],
                         add=True)

    def s_wait(ci, b):
        pltpu.make_async_copy(rows[b], acc_sh.at[didx_all.at[ci]],
                              ssem[b]).wait()

    for c0 in range(3):
        i_start(c0, c0)
    i_wait(0, 0)
    g_start(0, 0)

    def tri(p, _):
        for b in range(3):
            ci = 3 * p + b
            g_wait(ci, b)

            @pl.when(ci + 3 < NCHUNK)
            def _():
                i_start(ci + 3, b)

            s_start(ci, b)

            @pl.when(ci >= 2)
            def _():
                s_wait(ci - 2, (b + 1) % 3)

            @pl.when(ci + 1 < NCHUNK)
            def _():
                i_wait(ci + 1, (b + 1) % 3)
                g_start(ci + 1, (b + 1) % 3)
        return 0

    lax.fori_loop(0, NCHUNK // 3, tri, 0)
    c1 = (NCHUNK // 3) * 3
    c2 = c1 + 1
    g_wait(c1, 0)
    s_start(c1, 0)
    s_wait(c1 - 2, 1)
    i_wait(c2, 1)
    g_start(c2, 1)
    g_wait(c2, 1)
    s_start(c2, 1)
    s_wait(c1 - 1, 2)
    s_wait(c1, 0)
    s_wait(c2, 1)

    plsc.subcore_barrier()
    pltpu.sync_copy(acc_sh.at[pl.ds(rbase, NROWS_T)],
                    s_out.at[core, pl.ds(rbase, NROWS_T)])


def _make_combine(relu, out_dtype=jnp.float32):
    R = 1280
    grid = NPA // R

    def body(s_ref, c_ref, h_ref, wl_ref, b_ref, wr_ref, o_ref):
        cnt = jnp.sum(c_ref[...], axis=0)[:, None]
        inv = 1.0 / jnp.maximum(cnt, 1.0)
        agg = (s_ref[0] + s_ref[1]) * inv
        dn = (((1,), (1,)), ((), ()))
        y = (lax.dot_general(agg, wl_ref[...], dn,
                             preferred_element_type=jnp.float32)
             + b_ref[...]
             + lax.dot_general(h_ref[...], wr_ref[...], dn,
                               preferred_element_type=jnp.float32))
        y = jnp.maximum(y, 0.0) if relu else y
        o_ref[...] = y.astype(out_dtype)

    return pl.pallas_call(
        body,
        grid=(grid,),
        in_specs=[
            pl.BlockSpec((NC, R, D), lambda i: (0, i, 0)),
            pl.BlockSpec((NW, R), lambda i: (0, i)),
            pl.BlockSpec((R, D), lambda i: (i, 0)),
            pl.BlockSpec((D, D), lambda i: (0, 0)),
            pl.BlockSpec((1, D), lambda i: (0, 0)),
            pl.BlockSpec((D, D), lambda i: (0, 0)),
        ],
        out_specs=pl.BlockSpec((R, D), lambda i: (i, 0)),
        out_shape=jax.ShapeDtypeStruct((NPA, D), out_dtype),
    )


_combine_relu = _make_combine(True)
_combine_lin = _make_combine(False, jnp.bfloat16)


@functools.partial(
    pl.kernel,
    out_type=jax.ShapeDtypeStruct((E,), jnp.float32),
    mesh=_mesh,
    compiler_params=_cp,
    scratch_types=(
        [pltpu.VMEM((NCHUNK, K), jnp.int32) for _ in range(2)]
        + [pltpu.VMEM((K, D), jnp.int32) for _ in range(6)]
        + [pltpu.VMEM((K,), jnp.float32) for _ in range(2)]
        + [pltpu.VMEM((16, 16), jnp.float32)]
        + [pltpu.SemaphoreType.DMA for _ in range(8)]
    ),
)
def _decode(z_hbm, e03_hbm, e13_hbm, out_hbm, *rest):
    i0_all, i1_all = rest[0:2]
    av = rest[2:5]
    bv = rest[5:8]
    ov = rest[8:10]
    t_v = rest[10]
    gas = rest[11:14]
    gbs = rest[14:17]
    wss = rest[17:19]
    w = _wid()
    ebase = w * EPW
    lane = lax.iota(jnp.int32, 16)

    pltpu.sync_copy(e03_hbm.at[w], i0_all)
    pltpu.sync_copy(e13_hbm.at[w], i1_all)

    def g_start(ci, b):
        pltpu.async_copy(z_hbm.at[i0_all.at[ci]], av[b], gas[b])
        pltpu.async_copy(z_hbm.at[i1_all.at[ci]], bv[b], gbs[b])

    def g_wait(ci, b):
        pltpu.make_async_copy(z_hbm.at[i0_all.at[ci]], av[b], gas[b]).wait()
        pltpu.make_async_copy(z_hbm.at[i1_all.at[ci]], bv[b], gbs[b]).wait()

    def w_start(ci, b):
        off = ebase + ci * K
        pltpu.async_copy(ov[b], out_hbm.at[pl.ds(off, K)], wss[b])

    def w_wait(ci, b):
        off = ebase + ci * K
        pltpu.make_async_copy(ov[b], out_hbm.at[pl.ds(off, K)], wss[b]).wait()

    def compute(b3, b2):
        a_v = av[b3]
        b_v = bv[b3]
        o_v = ov[b2]

        def group(g, _):
            for l in range(16):
                e = g * 16 + l
                p = [plsc.bitcast(a_v[e, pl.ds(c * 16, 16)], jnp.bfloat16)
                     * plsc.bitcast(b_v[e, pl.ds(c * 16, 16)], jnp.bfloat16)
                     for c in range(4)]
                sb = (p[0] + p[1]) + (p[2] + p[3])
                lo, hi = plsc.unpack(sb, format=plsc.PackFormat.INTERLEAVED)
                acc = lo + hi
                plsc.store_scatter(t_v, [lane, jnp.full((16,), l, jnp.int32)],
                                   acc)
            res = t_v[0, :]
            for r in range(1, 16):
                res = res + t_v[r, :]
            o_v[pl.ds(g * 16, 16)] = res
            return 0

        lax.fori_loop(0, K // 16, group, 0)

    g_start(0, 0)
    g_start(1, 1)

    def six(p, _):
        for u in range(6):
            ci = 6 * p + u
            g_wait(ci, u % 3)
            g_start(ci + 2, (u + 2) % 3)

            @pl.when(ci >= 2)
            def _():
                w_wait(ci - 2, u % 2)

            compute(u % 3, u % 2)
            w_start(ci, u % 2)
        return 0

    lax.fori_loop(0, NCHUNK // 6, six, 0)
    c = (NCHUNK // 6) * 6
    for u in range(NCHUNK - c):
        ci = c + u
        g_wait(ci, ci % 3)
        if ci + 2 < NCHUNK:
            g_start(ci + 2, (ci + 2) % 3)
        w_wait(ci - 2, ci % 2)
        compute(ci % 3, ci % 2)
        w_start(ci, ci % 2)
    w_wait(NCHUNK - 2, (NCHUNK - 2) % 2)
    w_wait(NCHUNK - 1, (NCHUNK - 1) % 2)


def kernel(x, edge_index, edge_position, emb_table, W1l, b1, W1r, W2l, b2,
           W2r):
    src = edge_index[0]
    dst3 = edge_index[1].reshape(NW, NCHUNK, K)
    e03 = edge_position[0].reshape(NW, NCHUNK, K)
    e13 = edge_position[1].reshape(NW, NCHUNK, K)
    xp = jnp.concatenate([x[:, 0], jnp.zeros((NP - N,), jnp.int32)])
    z128 = jnp.zeros((NPA, D), jnp.float32)
    zn = jnp.zeros((NPA,), jnp.float32)
    b1r = b1.reshape(1, D)
    b2r = b2.reshape(1, D)

    h, c32 = _emb_gather(xp, emb_table, dst3, zn)
    s1 = _segsum(h, src, dst3, z128)
    h1 = _combine_relu(s1, c32, h, W1l, b1r, W1r)
    s2 = _segsum(h1, src, dst3, z128)
    z = _combine_lin(s2, c32, h1, W2l, b2r, W2r)
    z32 = lax.bitcast_convert_type(z.reshape(NPA, D // 2, 2), jnp.int32)
    z32p = jnp.concatenate(
        [z32, jnp.zeros((NPA, D // 2), jnp.int32)], axis=1)
    return _decode(z32p, e03, e13)


# speedup vs baseline: 1.0355x; 1.0355x over previous
import functools

import jax
import jax.numpy as jnp
from jax import lax
from jax.experimental import pallas as pl
from jax.experimental.pallas import tpu as pltpu
from jax.experimental.pallas import tpu_sc as plsc

N = 10000
E = 320000
D = 128

NC = 2
NS = 16
NW = NC * NS

NP = 10240
ROWS_W = NP // NW
EPW = E // NW
K = 80
NCHUNK = EPW // K
NPA = 10240
NROWS_T = NPA // NS

_mesh = plsc.VectorSubcoreMesh(core_axis_name="c", subcore_axis_name="s",
                               num_cores=NC, num_subcores=NS)
_cp = pltpu.CompilerParams(needs_layout_passes=False)


def _wid():
    return lax.axis_index("s") * NC + lax.axis_index("c")


@functools.partial(
    pl.kernel,
    out_type=[jax.ShapeDtypeStruct((NP, D), jnp.float32),
              jax.ShapeDtypeStruct((NW, NPA), jnp.float32)],
    mesh=_mesh,
    compiler_params=_cp,
    scratch_types=[
        pltpu.VMEM((K,), jnp.int32),
        pltpu.VMEM((K, D), jnp.float32),
        pltpu.VMEM((NCHUNK, K), jnp.int32),
        pltpu.VMEM((NPA,), jnp.float32),
        pltpu.SemaphoreType.DMA,
    ],
)
def _emb_gather(xp_hbm, emb_hbm, dst3_hbm, zn_hbm, h_out, c_out,
                idx_v, rows_v, didx_all, cnt_v, sem):
    w = _wid()
    base = w * ROWS_W

    def body(i, _):
        off = base + i * K
        pltpu.sync_copy(xp_hbm.at[pl.ds(off, K)], idx_v)
        pltpu.async_copy(emb_hbm.at[idx_v], rows_v, sem).wait()
        pltpu.sync_copy(rows_v, h_out.at[pl.ds(off, K)])
        return 0

    lax.fori_loop(0, ROWS_W // K, body, 0)

    pltpu.sync_copy(zn_hbm, cnt_v)
    pltpu.sync_copy(dst3_hbm.at[w], didx_all)
    ones = jnp.full((16,), 1.0, jnp.float32)

    def cbody(i, _):
        for j in range(K // 16):
            plsc.addupdate_scatter(cnt_v, [didx_all[i, pl.ds(j * 16, 16)]],
                                   ones)
        return 0

    lax.fori_loop(0, NCHUNK, cbody, 0)
    pltpu.sync_copy(cnt_v, c_out.at[w])


@functools.partial(
    pl.kernel,
    out_type=jax.ShapeDtypeStruct((NC, NPA, D), jnp.float32),
    mesh=_mesh,
    compiler_params=_cp,
    scratch_types=(
        [pltpu.VMEM((K,), jnp.int32) for _ in range(3)]
        + [pltpu.VMEM((NCHUNK, K), jnp.int32)]
        + [pltpu.VMEM((K, D), jnp.float32) for _ in range(3)]
        + [pltpu.VMEM_SHARED((NPA, D), jnp.float32)]
        + [pltpu.SemaphoreType.DMA for _ in range(9)]
    ),
)
def _segsum(h_hbm, src_hbm, dst3_hbm, z128_hbm, s_out, *rest):
    core = lax.axis_index("c")
    sub = lax.axis_index("s")
    rbase = sub * NROWS_T
    sidx = rest[0:3]
    didx_all = rest[3]
    rows = rest[4:7]
    acc_sh = rest[7]
    gsem = rest[8:11]
    ssem = rest[11:14]
    isem = rest[14:17]
    pltpu.sync_copy(z128_hbm.at[pl.ds(rbase, NROWS_T)],
                    acc_sh.at[pl.ds(rbase, NROWS_T)])
    w = _wid()
    pltpu.sync_copy(dst3_hbm.at[w], didx_all)
    plsc.subcore_barrier()

    ebase = w * EPW

    def i_start(ci, b):
        pltpu.async_copy(src_hbm.at[pl.ds(ebase + ci * K, K)], sidx[b],
                         isem[b])

    def i_wait(ci, b):
        pltpu.make_async_copy(src_hbm.at[pl.ds(ebase + ci * K, K)], sidx[b],
                              isem[b]).wait()

    def g_start(ci, b):
        pltpu.async_copy(h_hbm.at[sidx[b]], rows[b], gsem[b])

    def g_wait(ci, b):
        pltpu.make_async_copy(h_hbm.at[sidx[b]], rows[b], gsem[b]).wait()

    def s_start(ci, b):
        pltpu.async_copy(rows[b], acc_sh.at[didx_all.at[ci]], ssem[b],
                         add=True)

    def s_wait(ci, b):
        pltpu.make_async_copy(rows[b], acc_sh.at[didx_all.at[ci]],
                              ssem[b]).wait()

    for c0 in range(3):
        i_start(c0, c0)
    i_wait(0, 0)
    g_start(0, 0)

    def tri(p, _):
        for b in range(3):
            ci = 3 * p + b
            g_wait(ci, b)

            @pl.when(ci + 3 < NCHUNK)
            def _():
                i_start(ci + 3, b)

            s_start(ci, b)

            @pl.when(ci >= 2)
            def _():
                s_wait(ci - 2, (b + 1) % 3)

            @pl.when(ci + 1 < NCHUNK)
            def _():
                i_wait(ci + 1, (b + 1) % 3)
                g_start(ci + 1, (b + 1) % 3)
        return 0

    lax.fori_loop(0, NCHUNK // 3, tri, 0)
    c1 = (NCHUNK // 3) * 3
    c2 = c1 + 1
    g_wait(c1, 0)
    s_start(c1, 0)
    s_wait(c1 - 2, 1)
    i_wait(c2, 1)
    g_start(c2, 1)
    g_wait(c2, 1)
    s_start(c2, 1)
    s_wait(c1 - 1, 2)
    s_wait(c1, 0)
    s_wait(c2, 1)

    plsc.subcore_barrier()
    pltpu.sync_copy(acc_sh.at[pl.ds(rbase, NROWS_T)],
                    s_out.at[core, pl.ds(rbase, NROWS_T)])


def _make_combine(relu, out_dtype=jnp.float32):
    R = 1280
    grid = NPA // R

    def body(s_ref, c_ref, h_ref, wl_ref, b_ref, wr_ref, o_ref):
        cnt = jnp.sum(c_ref[...], axis=0)[:, None]
        inv = 1.0 / jnp.maximum(cnt, 1.0)
        agg = (s_ref[0] + s_ref[1]) * inv
        dn = (((1,), (1,)), ((), ()))
        y = (lax.dot_general(agg, wl_ref[...], dn,
                             preferred_element_type=jnp.float32)
             + b_ref[...]
             + lax.dot_general(h_ref[...], wr_ref[...], dn,
                               preferred_element_type=jnp.float32))
        y = jnp.maximum(y, 0.0) if relu else y
        o_ref[...] = y.astype(out_dtype)

    return pl.pallas_call(
        body,
        grid=(grid,),
        in_specs=[
            pl.BlockSpec((NC, R, D), lambda i: (0, i, 0)),
            pl.BlockSpec((NW, R), lambda i: (0, i)),
            pl.BlockSpec((R, D), lambda i: (i, 0)),
            pl.BlockSpec((D, D), lambda i: (0, 0)),
            pl.BlockSpec((1, D), lambda i: (0, 0)),
            pl.BlockSpec((D, D), lambda i: (0, 0)),
        ],
        out_specs=pl.BlockSpec((R, D), lambda i: (i, 0)),
        out_shape=jax.ShapeDtypeStruct((NPA, D), out_dtype),
    )


_combine_relu = _make_combine(True)
_combine_lin = _make_combine(False, jnp.bfloat16)


@functools.partial(
    pl.kernel,
    out_type=jax.ShapeDtypeStruct((E,), jnp.float32),
    mesh=_mesh,
    compiler_params=_cp,
    scratch_types=(
        [pltpu.VMEM((NCHUNK, K), jnp.int32) for _ in range(2)]
        + [pltpu.VMEM((K, D), jnp.int32) for _ in range(6)]
        + [pltpu.VMEM((K,), jnp.float32) for _ in range(2)]
        + [pltpu.VMEM((16, 16), jnp.int32)]
        + [pltpu.SemaphoreType.DMA for _ in range(8)]
    ),
)
def _decode(z_hbm, e03_hbm, e13_hbm, out_hbm, *rest):
    i0_all, i1_all = rest[0:2]
    av = rest[2:5]
    bv = rest[5:8]
    ov = rest[8:10]
    t_v = rest[10]
    gas = rest[11:14]
    gbs = rest[14:17]
    wss = rest[17:19]
    w = _wid()
    ebase = w * EPW
    lane = lax.iota(jnp.int32, 16)

    pltpu.sync_copy(e03_hbm.at[w], i0_all)
    pltpu.sync_copy(e13_hbm.at[w], i1_all)

    def g_start(ci, b):
        pltpu.async_copy(z_hbm.at[i0_all.at[ci]], av[b], gas[b])
        pltpu.async_copy(z_hbm.at[i1_all.at[ci]], bv[b], gbs[b])

    def g_wait(ci, b):
        pltpu.make_async_copy(z_hbm.at[i0_all.at[ci]], av[b], gas[b]).wait()
        pltpu.make_async_copy(z_hbm.at[i1_all.at[ci]], bv[b], gbs[b]).wait()

    def w_start(ci, b):
        off = ebase + ci * K
        pltpu.async_copy(ov[b], out_hbm.at[pl.ds(off, K)], wss[b])

    def w_wait(ci, b):
        off = ebase + ci * K
        pltpu.make_async_copy(ov[b], out_hbm.at[pl.ds(off, K)], wss[b]).wait()

    def compute(b3, b2):
        a_v = av[b3]
        b_v = bv[b3]
        o_v = ov[b2]

        def group(g, _):
            for l in range(16):
                e = g * 16 + l
                p = [plsc.bitcast(a_v[e, pl.ds(c * 16, 16)], jnp.bfloat16)
                     * plsc.bitcast(b_v[e, pl.ds(c * 16, 16)], jnp.bfloat16)
                     for c in range(4)]
                sb = (p[0] + p[1]) + (p[2] + p[3])
                plsc.store_scatter(t_v, [lane, jnp.full((16,), l, jnp.int32)],
                                   plsc.bitcast(sb, jnp.int32))
            q = [plsc.bitcast(t_v[r, :], jnp.bfloat16) for r in range(16)]
            while len(q) > 1:
                q = [q[i] + q[i + 1] for i in range(0, len(q), 2)]
            lo, hi = plsc.unpack(q[0], format=plsc.PackFormat.INTERLEAVED)
            o_v[pl.ds(g * 16, 16)] = lo + hi
            return 0

        lax.fori_loop(0, K // 16, group, 0)

    g_start(0, 0)
    g_start(1, 1)

    def six(p, _):
        for u in range(6):
            ci = 6 * p + u
            g_wait(ci, u % 3)
            g_start(ci + 2, (u + 2) % 3)

            @pl.when(ci >= 2)
            def _():
                w_wait(ci - 2, u % 2)

            compute(u % 3, u % 2)
            w_start(ci, u % 2)
        return 0

    lax.fori_loop(0, NCHUNK // 6, six, 0)
    c = (NCHUNK // 6) * 6
    for u in range(NCHUNK - c):
        ci = c + u
        g_wait(ci, ci % 3)
        if ci + 2 < NCHUNK:
            g_start(ci + 2, (ci + 2) % 3)
        w_wait(ci - 2, ci % 2)
        compute(ci % 3, ci % 2)
        w_start(ci, ci % 2)
    w_wait(NCHUNK - 2, (NCHUNK - 2) % 2)
    w_wait(NCHUNK - 1, (NCHUNK - 1) % 2)


def kernel(x, edge_index, edge_position, emb_table, W1l, b1, W1r, W2l, b2,
           W2r):
    src = edge_index[0]
    dst3 = edge_index[1].reshape(NW, NCHUNK, K)
    e03 = edge_position[0].reshape(NW, NCHUNK, K)
    e13 = edge_position[1].reshape(NW, NCHUNK, K)
    xp = jnp.concatenate([x[:, 0], jnp.zeros((NP - N,), jnp.int32)])
    z128 = jnp.zeros((NPA, D), jnp.float32)
    zn = jnp.zeros((NPA,), jnp.float32)
    b1r = b1.reshape(1, D)
    b2r = b2.reshape(1, D)

    h, c32 = _emb_gather(xp, emb_table, dst3, zn)
    s1 = _segsum(h, src, dst3, z128)
    h1 = _combine_relu(s1, c32, h, W1l, b1r, W1r)
    s2 = _segsum(h1, src, dst3, z128)
    z = _combine_lin(s2, c32, h1, W2l, b2r, W2r)
    z32 = lax.bitcast_convert_type(z.reshape(NPA, D // 2, 2), jnp.int32)
    z32p = jnp.concatenate(
        [z32, jnp.zeros((NPA, D // 2), jnp.int32)], axis=1)
    return _decode(z32p, e03, e13)
